# Initial kernel scaffold; baseline (speedup 1.0000x reference)
import jax
import jax.numpy as jnp
from jax.experimental import pallas as pl


def kernel(x, edge_index, edge_attr, node_emb, edge_emb, Wf1, bf1, Wf2, bf2, Wg1, bg1, Wg2, bg2, Wih, bih, Whh, bhh, Wn1, bn1, Wn2, bn2):
    raise NotImplementedError("write your pallas kernel here")


# speedup vs baseline: 2.3692x; 2.3692x over previous
import jax
import jax.numpy as jnp
from jax import lax
from jax.experimental import pallas as pl
from jax.experimental.pallas import tpu as pltpu
from jax.experimental.pallas import tpu_sc as plsc

N = 10000
E = 320000
D = 128
L = 5
NODE_TYPES = 10
EDGE_TYPES = 4

NC = 2
NS = 16
NW = NC * NS
EW = E // NW
CH = 80
NCH = EW // CH
NPAIR = (NCH + 1) // 2

NP = 10240
NB = NP // NS

BE = 1000
BN = 1000


def _sc_gather_body(h, src3, dst3, gs, gd,
                    idxs, idxd, sb0, sb1, db0, db1,
                    gss0, gss1, gsd0, gsd1, wss0, wss1, wsd0, wsd1):
  c = lax.axis_index("c")
  s = lax.axis_index("s")
  wid = s * NC + c
  base = wid * EW

  sbufs = (sb0, sb1)
  dbufs = (db0, db1)
  gsems_s = (gss0, gss1)
  gsems_d = (gsd0, gsd1)
  wsems_s = (wss0, wss1)
  wsems_d = (wsd0, wsd1)

  pltpu.sync_copy(src3.at[wid], idxs)
  pltpu.sync_copy(dst3.at[wid], idxd)

  def start_gather(j, b):
    pltpu.async_copy(h.at[idxs.at[j]], sbufs[b], gsems_s[b])
    pltpu.async_copy(h.at[idxd.at[j]], dbufs[b], gsems_d[b])

  def wait_gather(j, b):
    pltpu.make_async_copy(h.at[idxs.at[j]], sbufs[b], gsems_s[b]).wait()
    pltpu.make_async_copy(h.at[idxd.at[j]], dbufs[b], gsems_d[b]).wait()

  def start_wb(j, b):
    off = base + j * CH
    pltpu.async_copy(sbufs[b], gs.at[pl.ds(off, CH)], wsems_s[b])
    pltpu.async_copy(dbufs[b], gd.at[pl.ds(off, CH)], wsems_d[b])

  def wait_wb(b):
    pltpu.make_async_copy(sbufs[b], gs.at[pl.ds(base, CH)], wsems_s[b]).wait()
    pltpu.make_async_copy(dbufs[b], gd.at[pl.ds(base, CH)], wsems_d[b]).wait()

  start_gather(0, 0)

  def step(j, b):
    @pl.when(j + 1 < NCH)
    def _():
      @pl.when(j >= 1)
      def _():
        wait_wb(1 - b)
      start_gather(j + 1, 1 - b)

    wait_gather(j, b)
    start_wb(j, b)

  def pair(k, carry):
    j0 = k * 2
    step(j0, 0)

    @pl.when(j0 + 1 < NCH)
    def _():
      step(j0 + 1, 1)

    return carry

  lax.fori_loop(0, NPAIR, pair, 0)
  wait_wb(0)
  wait_wb(1)


def _sc_gather(h, src3, dst3):
  fn = pl.kernel(
      _sc_gather_body,
      out_type=(
          jax.ShapeDtypeStruct((E, D), jnp.float32),
          jax.ShapeDtypeStruct((E, D), jnp.float32),
      ),
      mesh=plsc.VectorSubcoreMesh(core_axis_name="c", subcore_axis_name="s"),
      scratch_types=[
          pltpu.VMEM((NCH, CH), jnp.int32),
          pltpu.VMEM((NCH, CH), jnp.int32),
          pltpu.VMEM((CH, D), jnp.float32),
          pltpu.VMEM((CH, D), jnp.float32),
          pltpu.VMEM((CH, D), jnp.float32),
          pltpu.VMEM((CH, D), jnp.float32),
      ] + [pltpu.SemaphoreType.DMA] * 8,
  )
  return fn(h, src3, dst3)


def _sc_scatter_body(ma, dst3, zrows, parts,
                     acc, idxd, rb0, rb1, ls0, ls1):
  c = lax.axis_index("c")
  s = lax.axis_index("s")
  wid = s * NC + c
  base = wid * EW

  rbufs = (rb0, rb1)
  lsems = (ls0, ls1)

  pltpu.sync_copy(zrows.at[pl.ds(0, NB)], acc.at[pl.ds(s * NB, NB)])
  pltpu.sync_copy(dst3.at[wid], idxd)
  plsc.subcore_barrier()

  def start_load(j, b):
    pltpu.async_copy(ma.at[pl.ds(base + j * CH, CH)], rbufs[b], lsems[b])

  def wait_load(b):
    pltpu.make_async_copy(ma.at[pl.ds(base, CH)], rbufs[b], lsems[b]).wait()

  start_load(0, 0)

  def step(j, b):
    @pl.when(j + 1 < NCH)
    def _():
      start_load(j + 1, 1 - b)

    wait_load(b)
    pltpu.sync_copy(rbufs[b], acc.at[idxd.at[j]], add=True)

  def pair(k, carry):
    j0 = k * 2
    step(j0, 0)

    @pl.when(j0 + 1 < NCH)
    def _():
      step(j0 + 1, 1)

    return carry

  lax.fori_loop(0, NPAIR, pair, 0)
  plsc.subcore_barrier()
  pltpu.sync_copy(acc.at[pl.ds(s * NB, NB)], parts.at[c, pl.ds(s * NB, NB)])


def _sc_scatter(ma, dst3, zrows):
  fn = pl.kernel(
      _sc_scatter_body,
      out_type=jax.ShapeDtypeStruct((NC, NP, D), jnp.float32),
      mesh=plsc.VectorSubcoreMesh(core_axis_name="c", subcore_axis_name="s"),
      scratch_types=[
          pltpu.VMEM_SHARED((NP, D), jnp.float32),
          pltpu.VMEM((NCH, CH), jnp.int32),
          pltpu.VMEM((CH, D), jnp.float32),
          pltpu.VMEM((CH, D), jnp.float32),
          pltpu.SemaphoreType.DMA,
          pltpu.SemaphoreType.DMA,
      ],
  )
  return fn(ma, dst3, zrows)


def _embed_body(xi_ref, emb_ref, out_ref):
  xi = xi_ref[0, 0, :]
  onehot = (xi[:, None] == lax.broadcasted_iota(jnp.int32, (BN, NODE_TYPES), 1))
  out_ref[...] = jnp.dot(onehot.astype(jnp.float32), emb_ref[...],
                         preferred_element_type=jnp.float32)


def _tc_embed(xi3, node_emb):
  return pl.pallas_call(
      _embed_body,
      grid=(N // BN,),
      in_specs=[
          pl.BlockSpec((1, 1, BN), lambda i: (i, 0, 0)),
          pl.BlockSpec((NODE_TYPES, D), lambda i: (0, 0)),
      ],
      out_specs=pl.BlockSpec((BN, D), lambda i: (i, 0)),
      out_shape=jax.ShapeDtypeStruct((N, D), jnp.float32),
  )(xi3, node_emb)


def _edge_body(gd_ref, gs_ref, ea_ref, emb_ref,
               wf1d_ref, wf1s_ref, wf1e_ref, bf1_ref,
               wg1d_ref, wg1s_ref, wg1e_ref, bg1_ref,
               wf2_ref, bf2_ref, wg2_ref, bg2_ref, out_ref):
  gd = gd_ref[...]
  gs = gs_ref[...]
  ea = ea_ref[0, 0, :]
  onehot = (ea[:, None] == lax.broadcasted_iota(jnp.int32, (BE, EDGE_TYPES), 1))
  onehot = onehot.astype(jnp.float32)
  emb = emb_ref[...]

  tf = jnp.dot(emb, wf1e_ref[...], preferred_element_type=jnp.float32) + bf1_ref[...]
  tg = jnp.dot(emb, wg1e_ref[...], preferred_element_type=jnp.float32) + bg1_ref[...]

  pre_f = (jnp.dot(gd, wf1d_ref[...], preferred_element_type=jnp.float32)
           + jnp.dot(gs, wf1s_ref[...], preferred_element_type=jnp.float32)
           + jnp.dot(onehot, tf, preferred_element_type=jnp.float32))
  pre_g = (jnp.dot(gd, wg1d_ref[...], preferred_element_type=jnp.float32)
           + jnp.dot(gs, wg1s_ref[...], preferred_element_type=jnp.float32)
           + jnp.dot(onehot, tg, preferred_element_type=jnp.float32))

  hf = jnp.maximum(pre_f, 0.0)
  hg = jnp.maximum(pre_g, 0.0)
  m = jnp.dot(hf, wf2_ref[...], preferred_element_type=jnp.float32) + bf2_ref[...]
  sgate = jnp.sum(hg * wg2_ref[...], axis=1, keepdims=True) + bg2_ref[...]
  a = jax.nn.sigmoid(sgate)
  out_ref[...] = m * a


def _tc_edge(gd, gs, ea3, edge_emb, wf1d, wf1s, wf1e, bf1r,
             wg1d, wg1s, wg1e, bg1r, wf2, bf2r, wg2r, bg2r):
  full = lambda shape: pl.BlockSpec(shape, lambda i: tuple(0 for _ in shape))
  return pl.pallas_call(
      _edge_body,
      grid=(E // BE,),
      in_specs=[
          pl.BlockSpec((BE, D), lambda i: (i, 0)),
          pl.BlockSpec((BE, D), lambda i: (i, 0)),
          pl.BlockSpec((1, 1, BE), lambda i: (i, 0, 0)),
          full((EDGE_TYPES, D)),
          full((D, D)), full((D, D)), full((D, D)), full((1, D)),
          full((D, D)), full((D, D)), full((D, D)), full((1, D)),
          full((D, D)), full((1, D)), full((1, D)), full((1, 1)),
      ],
      out_specs=pl.BlockSpec((BE, D), lambda i: (i, 0)),
      out_shape=jax.ShapeDtypeStruct((E, D), jnp.float32),
  )(gd, gs, ea3, edge_emb, wf1d, wf1s, wf1e, bf1r,
    wg1d, wg1s, wg1e, bg1r, wf2, bf2r, wg2r, bg2r)


def _gru_body(h_ref, p0_ref, p1_ref, wih_ref, bih_ref, whh_ref, bhh_ref, out_ref):
  h = h_ref[...]
  agg = p0_ref[0] + p1_ref[0]
  gi = jnp.dot(agg, wih_ref[...], preferred_element_type=jnp.float32) + bih_ref[...]
  gh = jnp.dot(h, whh_ref[...], preferred_element_type=jnp.float32) + bhh_ref[...]
  r = jax.nn.sigmoid(gi[:, 0:D] + gh[:, 0:D])
  z = jax.nn.sigmoid(gi[:, D:2 * D] + gh[:, D:2 * D])
  n = jnp.tanh(gi[:, 2 * D:3 * D] + r * gh[:, 2 * D:3 * D])
  out_ref[...] = (1.0 - z) * n + z * h


def _tc_gru(h, parts, wih, bihr, whh, bhhr):
  full = lambda shape: pl.BlockSpec(shape, lambda i: tuple(0 for _ in shape))
  return pl.pallas_call(
      _gru_body,
      grid=(N // BN,),
      in_specs=[
          pl.BlockSpec((BN, D), lambda i: (i, 0)),
          pl.BlockSpec((1, BN, D), lambda i: (0, i, 0)),
          pl.BlockSpec((1, BN, D), lambda i: (1, i, 0)),
          full((D, 3 * D)), full((1, 3 * D)),
          full((D, 3 * D)), full((1, 3 * D)),
      ],
      out_specs=pl.BlockSpec((BN, D), lambda i: (i, 0)),
      out_shape=jax.ShapeDtypeStruct((N, D), jnp.float32),
  )(h, parts, parts, wih, bihr, whh, bhhr)


def _mean_body(h_ref, out_ref):
  @pl.when(pl.program_id(0) == 0)
  def _():
    out_ref[...] = jnp.zeros((1, D), jnp.float32)

  out_ref[...] += jnp.sum(h_ref[...], axis=0, keepdims=True) * (1.0 / N)


def _tc_mean(h):
  return pl.pallas_call(
      _mean_body,
      grid=(N // BN,),
      in_specs=[pl.BlockSpec((BN, D), lambda i: (i, 0))],
      out_specs=pl.BlockSpec((1, D), lambda i: (0, 0)),
      out_shape=jax.ShapeDtypeStruct((1, D), jnp.float32),
  )(h)


def _readout_body(h_ref, hg_ref, wn1a_ref, wn1b_ref, bn1_ref, wn2_ref, bn2_ref,
                  out_ref):
  c0 = jnp.dot(hg_ref[...], wn1a_ref[...], preferred_element_type=jnp.float32)
  z = jnp.maximum(
      jnp.dot(h_ref[...], wn1b_ref[...], preferred_element_type=jnp.float32)
      + c0 + bn1_ref[...], 0.0)
  logits = jnp.dot(z, wn2_ref[...], preferred_element_type=jnp.float32) + bn2_ref[...]
  mx = jnp.max(logits, axis=1, keepdims=True)
  p = jnp.exp(logits - mx)
  out_ref[...] = p / jnp.sum(p, axis=1, keepdims=True)


def _tc_readout(h, hg, wn1a, wn1b, bn1r, wn2, bn2r):
  full = lambda shape: pl.BlockSpec(shape, lambda i: tuple(0 for _ in shape))
  return pl.pallas_call(
      _readout_body,
      grid=(N // BN,),
      in_specs=[
          pl.BlockSpec((BN, D), lambda i: (i, 0)),
          full((1, D)),
          full((D, D)), full((D, D)), full((1, D)),
          full((D, NODE_TYPES)), full((1, NODE_TYPES)),
      ],
      out_specs=pl.BlockSpec((BN, NODE_TYPES), lambda i: (i, 0)),
      out_shape=jax.ShapeDtypeStruct((N, NODE_TYPES), jnp.float32),
  )(h, hg, wn1a, wn1b, bn1r, wn2, bn2r)


def kernel(x, edge_index, edge_attr, node_emb, edge_emb, Wf1, bf1, Wf2, bf2,
           Wg1, bg1, Wg2, bg2, Wih, bih, Whh, bhh, Wn1, bn1, Wn2, bn2):
  xi3 = x.astype(jnp.int32).reshape(N // BN, 1, BN)
  ea3 = edge_attr.astype(jnp.int32).reshape(E // BE, 1, BE)
  src3 = edge_index[0].astype(jnp.int32).reshape(NW, NCH, CH)
  dst3 = edge_index[1].astype(jnp.int32).reshape(NW, NCH, CH)
  zrows = jnp.zeros((NB, D), jnp.float32)

  h = _tc_embed(xi3, node_emb)

  for l in range(L):
    gs, gd = _sc_gather(h, src3, dst3)
    ma = _tc_edge(
        gd, gs, ea3, edge_emb,
        Wf1[l, 0:D, :], Wf1[l, D:2 * D, :], Wf1[l, 2 * D:3 * D, :],
        bf1[l][None, :],
        Wg1[l, 0:D, :], Wg1[l, D:2 * D, :], Wg1[l, 2 * D:3 * D, :],
        bg1[l][None, :],
        Wf2[l], bf2[l][None, :], Wg2[l].reshape(1, D), bg2[l][None, :])
    parts = _sc_scatter(ma, dst3, zrows)
    h = _tc_gru(h, parts, Wih[l], bih[l][None, :], Whh[l], bhh[l][None, :])

  hg = _tc_mean(h)
  return _tc_readout(h, hg, Wn1[0:D, :], Wn1[D:2 * D, :], bn1[None, :],
                     Wn2, bn2[None, :])
